# baseline (speedup 1.0000x reference)
import functools

import jax
import jax.numpy as jnp
from jax.experimental import pallas as pl
from jax.experimental.pallas import tpu as pltpu


def _moe_body(x_ref, pc_ref, w1_ref, b1_ref, w2_ref, b2_ref, w3_ref, b3_ref,
              out_ref, h2_ref):
    e = pl.program_id(1)
    hb = pl.program_id(2)
    n_hb = pl.num_programs(2)

    h1 = jnp.dot(x_ref[...], w1_ref[0], preferred_element_type=jnp.float32)
    h1 = jnp.maximum(h1 + b1_ref[0].astype(jnp.float32), 0.0)
    part = jnp.dot(h1.astype(jnp.bfloat16), w2_ref[0],
                   preferred_element_type=jnp.float32)

    @pl.when(hb == 0)
    def _():
        h2_ref[...] = part

    @pl.when(hb != 0)
    def _():
        h2_ref[...] += part

    @pl.when(hb == n_hb - 1)
    def _():
        h2r = jnp.maximum(h2_ref[...] + b2_ref[0].astype(jnp.float32), 0.0)
        z = jnp.dot(h2r, w3_ref[0], preferred_element_type=jnp.float32)
        expert_out = jax.nn.sigmoid(z + b3_ref[0]) * pc_ref[0]

        @pl.when(e == 0)
        def _():
            out_ref[...] = expert_out

        @pl.when(e != 0)
        def _():
            out_ref[...] += expert_out


@functools.partial(jax.jit, static_argnames=())
def kernel(x, soft_cluster_probs, W1, b1, W2, b2, W3, b3):
    B, D = x.shape
    E, _, H = W1.shape
    H2 = W2.shape[2]
    O = W3.shape[2]

    TB = min(1024, B)
    HB = min(512, H)
    grid = (B // TB, E, H // HB)

    xb = x.astype(jnp.bfloat16)
    W1b = W1.astype(jnp.bfloat16)
    W2b = W2.astype(jnp.bfloat16)
    pc = soft_cluster_probs.T[:, :, None]
    b1r = b1[:, None, :]
    b2r = b2[:, None, :]
    b3r = b3[:, None, :]

    out = pl.pallas_call(
        _moe_body,
        grid=grid,
        in_specs=[
            pl.BlockSpec((TB, D), lambda t, e, h: (t, 0)),
            pl.BlockSpec((1, TB, 1), lambda t, e, h: (e, t, 0)),
            pl.BlockSpec((1, D, HB), lambda t, e, h: (e, 0, h)),
            pl.BlockSpec((1, 1, HB), lambda t, e, h: (e, 0, h)),
            pl.BlockSpec((1, HB, H2), lambda t, e, h: (e, h, 0)),
            pl.BlockSpec((1, 1, H2), lambda t, e, h: (e, 0, 0)),
            pl.BlockSpec((1, H2, O), lambda t, e, h: (e, 0, 0)),
            pl.BlockSpec((1, 1, O), lambda t, e, h: (e, 0, 0)),
        ],
        out_specs=pl.BlockSpec((TB, O), lambda t, e, h: (t, 0)),
        out_shape=jax.ShapeDtypeStruct((B, O), jnp.float32),
        scratch_shapes=[pltpu.VMEM((TB, H2), jnp.float32)],
        compiler_params=pltpu.CompilerParams(
            dimension_semantics=("parallel", "arbitrary", "arbitrary")),
    )(xb, pc, W1b, b1r, W2b, b2r, W3, b3r)
    return out


# speedup vs baseline: 1.0371x; 1.0371x over previous
import functools

import jax
import jax.numpy as jnp
from jax.experimental import pallas as pl
from jax.experimental.pallas import tpu as pltpu


def _moe_body(x_ref, pc_ref, w1_ref, b1_ref, w2_ref, b2_ref, w3_ref, b3_ref,
              out_ref, h2_ref):
    e = pl.program_id(1)
    hb = pl.program_id(2)
    n_hb = pl.num_programs(2)

    h1 = jnp.dot(x_ref[...], w1_ref[0], preferred_element_type=jnp.float32)
    h1 = jnp.maximum(h1 + b1_ref[0].astype(jnp.float32), 0.0).astype(jnp.bfloat16)
    part = jnp.dot(h1, w2_ref[0], preferred_element_type=jnp.float32)

    @pl.when(hb == 0)
    def _():
        h2_ref[...] = part.astype(jnp.bfloat16)

    @pl.when(hb != 0)
    def _():
        h2_ref[...] += part.astype(jnp.bfloat16)

    @pl.when(hb == n_hb - 1)
    def _():
        h2r = jnp.maximum(h2_ref[...] + b2_ref[0], 0.0)
        z = jnp.dot(h2r, w3_ref[0], preferred_element_type=jnp.float32)
        expert_out = jax.nn.sigmoid(z + b3_ref[0]) * pc_ref[0]

        @pl.when(e == 0)
        def _():
            out_ref[...] = expert_out

        @pl.when(e != 0)
        def _():
            out_ref[...] += expert_out


@functools.partial(jax.jit, static_argnames=())
def kernel(x, soft_cluster_probs, W1, b1, W2, b2, W3, b3):
    B, D = x.shape
    E, _, H = W1.shape
    H2 = W2.shape[2]
    O = W3.shape[2]

    TB = min(1024, B)
    HB = min(512, H)
    grid = (B // TB, E, H // HB)

    xb = x.astype(jnp.bfloat16)
    W1b = W1.astype(jnp.bfloat16)
    W2b = W2.astype(jnp.bfloat16)
    pc = soft_cluster_probs.T[:, :, None]
    W3b = W3.astype(jnp.bfloat16)
    b1r = b1[:, None, :].astype(jnp.bfloat16)
    b2r = b2[:, None, :].astype(jnp.bfloat16)
    b3r = b3[:, None, :]

    out = pl.pallas_call(
        _moe_body,
        grid=grid,
        in_specs=[
            pl.BlockSpec((TB, D), lambda t, e, h: (t, 0)),
            pl.BlockSpec((1, TB, 1), lambda t, e, h: (e, t, 0)),
            pl.BlockSpec((1, D, HB), lambda t, e, h: (e, 0, h)),
            pl.BlockSpec((1, 1, HB), lambda t, e, h: (e, 0, h)),
            pl.BlockSpec((1, HB, H2), lambda t, e, h: (e, h, 0)),
            pl.BlockSpec((1, 1, H2), lambda t, e, h: (e, 0, 0)),
            pl.BlockSpec((1, H2, O), lambda t, e, h: (e, 0, 0)),
            pl.BlockSpec((1, 1, O), lambda t, e, h: (e, 0, 0)),
        ],
        out_specs=pl.BlockSpec((TB, O), lambda t, e, h: (t, 0)),
        out_shape=jax.ShapeDtypeStruct((B, O), jnp.float32),
        scratch_shapes=[pltpu.VMEM((TB, H2), jnp.bfloat16)],
        compiler_params=pltpu.CompilerParams(
            dimension_semantics=("parallel", "arbitrary", "arbitrary")),
    )(xb, pc, W1b, b1r, W2b, b2r, W3b, b3r)
    return out
